# baseline (speedup 1.0000x reference)
import functools

import jax
import jax.numpy as jnp
from jax import lax
from jax.experimental import pallas as pl
from jax.experimental.pallas import tpu as pltpu
from jax.experimental.pallas import tpu_sc as plsc

N = 4096
E = 262144
NC, NS = 2, 16
NW = NC * NS
EPT = E // NW
CHUNK = 128
CH = EPT // CHUNK
GRP = 8


def _make_edge_pass(F):
    mesh = plsc.VectorSubcoreMesh(core_axis_name="c", subcore_axis_name="s")
    rpt = N // NS

    @functools.partial(
        pl.kernel,
        out_type=jax.ShapeDtypeStruct((NC, N, F), jnp.int32),
        mesh=mesh,
        compiler_params=pltpu.CompilerParams(use_tc_tiling_on_sc=False),
        scratch_types=[
            pltpu.VMEM((CH, CHUNK), jnp.int32),
            pltpu.VMEM((CH, CHUNK), jnp.int32),
            pltpu.VMEM((GRP, CHUNK, F), jnp.int32),
            pltpu.VMEM_SHARED((N, F), jnp.int32),
            pltpu.SemaphoreType.DMA,
            pltpu.SemaphoreType.DMA,
        ],
    )
    def edge_pass(table_hbm, src_hbm, dst_hbm, zeros_hbm, out_hbm,
                  src_v, dst_v, msg_v, acc_sh, gsem, ssem):
        cid = lax.axis_index("c")
        sid = lax.axis_index("s")
        wid = cid * NS + sid

        pltpu.sync_copy(src_hbm.at[wid], src_v)
        pltpu.sync_copy(dst_hbm.at[wid], dst_v)
        pltpu.sync_copy(zeros_hbm.at[pl.ds(sid * rpt, rpt)],
                        acc_sh.at[pl.ds(sid * rpt, rpt)])
        plsc.subcore_barrier()

        def group(g, carry):
            gds = []
            for i in range(GRP):
                j = g * GRP + i
                gds.append(pltpu.async_copy(
                    table_hbm.at[src_v.at[j]], msg_v.at[i], gsem))
            sds = []
            for i in range(GRP):
                j = g * GRP + i
                gds[i].wait()
                sds.append(pltpu.async_copy(
                    msg_v.at[i], acc_sh.at[dst_v.at[j]], ssem, add=True))
            for d in sds:
                d.wait()
            return carry

        lax.fori_loop(0, CH // GRP, group, 0, unroll=False)
        plsc.subcore_barrier()

        pltpu.sync_copy(acc_sh.at[pl.ds(sid * rpt, rpt)],
                        out_hbm.at[cid].at[pl.ds(sid * rpt, rpt)])

    return edge_pass


FP = 8
_edge_pass_8 = _make_edge_pass(FP)
_FXS = 2.0 ** 20


def _fx_scatter(p, bound, src3, dst3):
    F = p.shape[1]
    s1 = jnp.exp2(23.0 - jnp.ceil(jnp.log2(jnp.maximum(bound, 1e-30))))
    ps = p * s1
    hi = jnp.round(ps)
    lo = jnp.round((ps - hi) * _FXS)
    pad = jnp.zeros((N, 4 - F), jnp.float32)
    table = jnp.concatenate([hi, pad, lo, pad], axis=1).astype(jnp.int32)
    parts = _edge_pass_8(table, src3, dst3, jnp.zeros((N, FP), jnp.int32))
    hs = (parts[0, :, :F] + parts[1, :, :F]).astype(jnp.float32)
    ls = (parts[0, :, 4:4 + F] + parts[1, :, 4:4 + F]).astype(jnp.float32)
    return (hs + ls * (1.0 / _FXS)) * (1.0 / s1)


_BJ = 512
_BI = 1024


def _lw1_body(ga_ref, gc_ref, aw_ref, ab_ref, cw_ref, cb_ref, ta_ref, tc_ref):
    ta_ref[...] = jnp.tanh(
        jnp.dot(ga_ref[...], aw_ref[...], preferred_element_type=jnp.float32, precision=lax.Precision.HIGHEST)
        + ab_ref[...])
    tc_ref[...] = jnp.tanh(
        jnp.dot(gc_ref[...], cw_ref[...], preferred_element_type=jnp.float32, precision=lax.Precision.HIGHEST)
        + cb_ref[...])


def _head_layer1(ga, gc, a_lw1, a_lb1, c_lw1, c_lb1):
    H = a_lw1.shape[1]
    grid = (H // _BJ,)
    return pl.pallas_call(
        _lw1_body,
        grid=grid,
        in_specs=[
            pl.BlockSpec((1, N), lambda j: (0, 0)),
            pl.BlockSpec((1, N), lambda j: (0, 0)),
            pl.BlockSpec((N, _BJ), lambda j: (0, j)),
            pl.BlockSpec((1, _BJ), lambda j: (0, j)),
            pl.BlockSpec((N, _BJ), lambda j: (0, j)),
            pl.BlockSpec((1, _BJ), lambda j: (0, j)),
        ],
        out_specs=[pl.BlockSpec((1, _BJ), lambda j: (0, j))] * 2,
        out_shape=[jax.ShapeDtypeStruct((1, H), jnp.float32)] * 2,
    )(ga, gc, a_lw1, a_lb1.reshape(1, -1), c_lw1, c_lb1.reshape(1, -1))


def _lw2_body(ta_ref, tc_ref, aw_ref, ab_ref, cw_ref, cb_ref,
              probs_ref, val_ref, act_ref, acc_ref, vacc_ref):
    i = pl.program_id(0)
    ni = pl.num_programs(0)

    @pl.when(i == 0)
    def _():
        acc_ref[...] = jnp.zeros_like(acc_ref)
        vacc_ref[...] = jnp.zeros_like(vacc_ref)

    acc_ref[...] += jnp.dot(ta_ref[...], aw_ref[...],
                            preferred_element_type=jnp.float32,
                            precision=lax.Precision.HIGHEST)
    vacc_ref[...] += jnp.dot(tc_ref[...], cw_ref[...],
                             preferred_element_type=jnp.float32,
                             precision=lax.Precision.HIGHEST)

    @pl.when(i == ni - 1)
    def _():
        outa = jnp.tanh(acc_ref[...] + ab_ref[...])
        m = jnp.max(outa)
        e = jnp.exp(outa - m)
        probs_ref[...] = e / jnp.sum(e)
        val_ref[...] = jnp.tanh(vacc_ref[...] + cb_ref[...])
        idx = lax.broadcasted_iota(jnp.int32, outa.shape, 1)
        act_ref[0, 0] = jnp.min(jnp.where(outa == m, idx, jnp.int32(2**30)))


def _head_layer2(t1a, t1c, a_lw2, a_lb2, c_lw2, c_lb2):
    H = a_lw2.shape[0]
    grid = (H // _BI,)
    return pl.pallas_call(
        _lw2_body,
        grid=grid,
        in_specs=[
            pl.BlockSpec((1, _BI), lambda i: (0, i)),
            pl.BlockSpec((1, _BI), lambda i: (0, i)),
            pl.BlockSpec((_BI, N), lambda i: (i, 0)),
            pl.BlockSpec((1, N), lambda i: (0, 0)),
            pl.BlockSpec((_BI, 1), lambda i: (i, 0)),
            pl.BlockSpec((1, 1), lambda i: (0, 0)),
        ],
        out_specs=[
            pl.BlockSpec((1, N), lambda i: (0, 0)),
            pl.BlockSpec((1, 1), lambda i: (0, 0)),
            pl.BlockSpec(memory_space=pltpu.MemorySpace.SMEM),
        ],
        out_shape=[
            jax.ShapeDtypeStruct((1, N), jnp.float32),
            jax.ShapeDtypeStruct((1, 1), jnp.float32),
            jax.ShapeDtypeStruct((1, 1), jnp.int32),
        ],
        scratch_shapes=[
            pltpu.VMEM((1, N), jnp.float32),
            pltpu.VMEM((1, 1), jnp.float32),
        ],
    )(t1a, t1c, a_lw2, a_lb2.reshape(1, -1), c_lw2, c_lb2.reshape(1, -1))


def kernel(data, edge_index, a_cw1, a_cb1, a_cw2, a_cb2, a_cw3, a_cb3,
           a_lw1, a_lb1, a_lw2, a_lb2, c_cw1, c_cb1, c_cw2, c_cb2,
           c_cw3, c_cb3, c_lw1, c_lb1, c_lw2, c_lb2):
    ei = edge_index.astype(jnp.int32)
    src3 = ei[0].reshape(NW, CH, CHUNK)
    dst3 = ei[1].reshape(NW, CH, CHUNK)

    deg_tab = jnp.concatenate(
        [jnp.ones((N, 1), jnp.int32), jnp.zeros((N, FP - 1), jnp.int32)], axis=1)
    dparts = _edge_pass_8(deg_tab, dst3, dst3, jnp.zeros((N, FP), jnp.int32))
    deg = 1.0 + (dparts[0, :, :1] + dparts[1, :, :1]).astype(jnp.float32)
    dinv = lax.rsqrt(deg)
    deg_max = jnp.max(deg)

    x = data
    p1 = dinv * x
    b1 = deg_max * jnp.max(jnp.abs(p1))
    z1 = dinv * (_fx_scatter(p1, b1, src3, dst3) + dinv * p1)
    h1a = jax.nn.relu(z1 @ a_cw1 + a_cb1)
    h1c = jax.nn.relu(z1 @ c_cw1 + c_cb1)

    q2 = jnp.concatenate([h1a @ a_cw2, h1c @ c_cw2], axis=1)
    p2 = dinv * q2
    b2 = deg_max * jnp.max(jnp.abs(p2))
    z2 = dinv * (_fx_scatter(p2, b2, src3, dst3) + dinv * p2)
    h2a = jax.nn.relu(z2[:, 0:2] + a_cb2)
    h2c = jax.nn.relu(z2[:, 2:4] + c_cb2)

    q3 = jnp.concatenate([h2a @ a_cw3, h2c @ c_cw3], axis=1)
    p3 = dinv * q3
    b3 = deg_max * jnp.max(jnp.abs(p3))
    z3 = dinv * (_fx_scatter(p3, b3, src3, dst3) + dinv * p3)
    ga = (z3[:, 0:1] + a_cb3).reshape(1, N)
    gc = (z3[:, 1:2] + c_cb3).reshape(1, N)

    t1a, t1c = _head_layer1(ga, gc, a_lw1, a_lb1, c_lw1, c_lb1)
    probs, value, act = _head_layer2(t1a, t1c, a_lw2, a_lb2, c_lw2, c_lb2)
    return (probs, value, act.reshape(()).astype(jnp.int32))


# speedup vs baseline: 1.1354x; 1.1354x over previous
import functools

import jax
import jax.numpy as jnp
from jax import lax
from jax.experimental import pallas as pl
from jax.experimental.pallas import tpu as pltpu
from jax.experimental.pallas import tpu_sc as plsc

N = 4096
E = 262144
NC, NS = 2, 16
NW = NC * NS
EPT = E // NW
CHUNK = 128
CH = EPT // CHUNK
GRP = 8


def _make_edge_pass(F):
    mesh = plsc.VectorSubcoreMesh(core_axis_name="c", subcore_axis_name="s")
    rpt = N // NS

    @functools.partial(
        pl.kernel,
        out_type=jax.ShapeDtypeStruct((NC, N, F), jnp.int32),
        mesh=mesh,
        compiler_params=pltpu.CompilerParams(use_tc_tiling_on_sc=False),
        scratch_types=[
            pltpu.VMEM((CH, CHUNK), jnp.int32),
            pltpu.VMEM((CH, CHUNK), jnp.int32),
            pltpu.VMEM((GRP, CHUNK, F), jnp.int32),
            pltpu.VMEM_SHARED((N, F), jnp.int32),
            pltpu.VMEM_SHARED((N, F), jnp.int32),
            pltpu.SemaphoreType.DMA,
            pltpu.SemaphoreType.DMA,
        ],
    )
    def edge_pass(table_hbm, src_hbm, dst_hbm, zeros_hbm, out_hbm,
                  src_v, dst_v, msg_v, tab_sh, acc_sh, gsem, ssem):
        cid = lax.axis_index("c")
        sid = lax.axis_index("s")
        wid = cid * NS + sid

        pltpu.sync_copy(src_hbm.at[wid], src_v)
        pltpu.sync_copy(dst_hbm.at[wid], dst_v)
        pltpu.sync_copy(table_hbm.at[pl.ds(sid * rpt, rpt)],
                        tab_sh.at[pl.ds(sid * rpt, rpt)])
        pltpu.sync_copy(zeros_hbm.at[pl.ds(sid * rpt, rpt)],
                        acc_sh.at[pl.ds(sid * rpt, rpt)])
        plsc.subcore_barrier()

        def group(g, carry):
            gds = []
            for i in range(GRP):
                j = g * GRP + i
                gds.append(pltpu.async_copy(
                    tab_sh.at[src_v.at[j]], msg_v.at[i], gsem))
            sds = []
            for i in range(GRP):
                j = g * GRP + i
                gds[i].wait()
                sds.append(pltpu.async_copy(
                    msg_v.at[i], acc_sh.at[dst_v.at[j]], ssem, add=True))
            for d in sds:
                d.wait()
            return carry

        lax.fori_loop(0, CH // GRP, group, 0, unroll=False)
        plsc.subcore_barrier()

        pltpu.sync_copy(acc_sh.at[pl.ds(sid * rpt, rpt)],
                        out_hbm.at[cid].at[pl.ds(sid * rpt, rpt)])

    return edge_pass


FP = 8
_edge_pass_8 = _make_edge_pass(FP)
_FXS = 2.0 ** 20


def _fx_scatter(p, bound, src3, dst3):
    F = p.shape[1]
    s1 = jnp.exp2(23.0 - jnp.ceil(jnp.log2(jnp.maximum(bound, 1e-30))))
    ps = p * s1
    hi = jnp.round(ps)
    lo = jnp.round((ps - hi) * _FXS)
    pad = jnp.zeros((N, 4 - F), jnp.float32)
    table = jnp.concatenate([hi, pad, lo, pad], axis=1).astype(jnp.int32)
    parts = _edge_pass_8(table, src3, dst3, jnp.zeros((N, FP), jnp.int32))
    hs = (parts[0, :, :F] + parts[1, :, :F]).astype(jnp.float32)
    ls = (parts[0, :, 4:4 + F] + parts[1, :, 4:4 + F]).astype(jnp.float32)
    return (hs + ls * (1.0 / _FXS)) * (1.0 / s1)


_BJ = 512
_BI = 512


def _lw1_body(ga_ref, gc_ref, aw_ref, ab_ref, cw_ref, cb_ref, ta_ref, tc_ref):
    ta_ref[...] = jnp.tanh(
        jnp.dot(ga_ref[...], aw_ref[...], preferred_element_type=jnp.float32, precision=lax.Precision.HIGHEST)
        + ab_ref[...])
    tc_ref[...] = jnp.tanh(
        jnp.dot(gc_ref[...], cw_ref[...], preferred_element_type=jnp.float32, precision=lax.Precision.HIGHEST)
        + cb_ref[...])


def _head_layer1(ga, gc, a_lw1, a_lb1, c_lw1, c_lb1):
    H = a_lw1.shape[1]
    grid = (H // _BJ,)
    return pl.pallas_call(
        _lw1_body,
        grid=grid,
        in_specs=[
            pl.BlockSpec((1, N), lambda j: (0, 0)),
            pl.BlockSpec((1, N), lambda j: (0, 0)),
            pl.BlockSpec((N, _BJ), lambda j: (0, j)),
            pl.BlockSpec((1, _BJ), lambda j: (0, j)),
            pl.BlockSpec((N, _BJ), lambda j: (0, j)),
            pl.BlockSpec((1, _BJ), lambda j: (0, j)),
        ],
        out_specs=[pl.BlockSpec((1, _BJ), lambda j: (0, j))] * 2,
        out_shape=[jax.ShapeDtypeStruct((1, H), jnp.float32)] * 2,
    )(ga, gc, a_lw1, a_lb1.reshape(1, -1), c_lw1, c_lb1.reshape(1, -1))


def _lw2_body(ta_ref, tc_ref, aw_ref, ab_ref, cw_ref, cb_ref,
              probs_ref, val_ref, act_ref, acc_ref, vacc_ref):
    i = pl.program_id(0)
    ni = pl.num_programs(0)

    @pl.when(i == 0)
    def _():
        acc_ref[...] = jnp.zeros_like(acc_ref)
        vacc_ref[...] = jnp.zeros_like(vacc_ref)

    acc_ref[...] += jnp.dot(ta_ref[...], aw_ref[...],
                            preferred_element_type=jnp.float32,
                            precision=lax.Precision.HIGHEST)
    vacc_ref[...] += jnp.dot(tc_ref[...], cw_ref[...],
                             preferred_element_type=jnp.float32,
                             precision=lax.Precision.HIGHEST)

    @pl.when(i == ni - 1)
    def _():
        outa = jnp.tanh(acc_ref[...] + ab_ref[...])
        m = jnp.max(outa)
        e = jnp.exp(outa - m)
        probs_ref[...] = e / jnp.sum(e)
        val_ref[...] = jnp.tanh(vacc_ref[...] + cb_ref[...])
        idx = lax.broadcasted_iota(jnp.int32, outa.shape, 1)
        act_ref[0, 0] = jnp.min(jnp.where(outa == m, idx, jnp.int32(2**30)))


def _head_layer2(t1a, t1c, a_lw2, a_lb2, c_lw2, c_lb2):
    H = a_lw2.shape[0]
    grid = (H // _BI,)
    return pl.pallas_call(
        _lw2_body,
        grid=grid,
        in_specs=[
            pl.BlockSpec((1, _BI), lambda i: (0, i)),
            pl.BlockSpec((1, _BI), lambda i: (0, i)),
            pl.BlockSpec((_BI, N), lambda i: (i, 0)),
            pl.BlockSpec((1, N), lambda i: (0, 0)),
            pl.BlockSpec((_BI, 1), lambda i: (i, 0)),
            pl.BlockSpec((1, 1), lambda i: (0, 0)),
        ],
        out_specs=[
            pl.BlockSpec((1, N), lambda i: (0, 0)),
            pl.BlockSpec((1, 1), lambda i: (0, 0)),
            pl.BlockSpec(memory_space=pltpu.MemorySpace.SMEM),
        ],
        out_shape=[
            jax.ShapeDtypeStruct((1, N), jnp.float32),
            jax.ShapeDtypeStruct((1, 1), jnp.float32),
            jax.ShapeDtypeStruct((1, 1), jnp.int32),
        ],
        scratch_shapes=[
            pltpu.VMEM((1, N), jnp.float32),
            pltpu.VMEM((1, 1), jnp.float32),
        ],
    )(t1a, t1c, a_lw2, a_lb2.reshape(1, -1), c_lw2, c_lb2.reshape(1, -1))


def kernel(data, edge_index, a_cw1, a_cb1, a_cw2, a_cb2, a_cw3, a_cb3,
           a_lw1, a_lb1, a_lw2, a_lb2, c_cw1, c_cb1, c_cw2, c_cb2,
           c_cw3, c_cb3, c_lw1, c_lb1, c_lw2, c_lb2):
    ei = edge_index.astype(jnp.int32)
    src3 = ei[0].reshape(NW, CH, CHUNK)
    dst3 = ei[1].reshape(NW, CH, CHUNK)

    deg_tab = jnp.concatenate(
        [jnp.ones((N, 1), jnp.int32), jnp.zeros((N, FP - 1), jnp.int32)], axis=1)
    dparts = _edge_pass_8(deg_tab, dst3, dst3, jnp.zeros((N, FP), jnp.int32))
    deg = 1.0 + (dparts[0, :, :1] + dparts[1, :, :1]).astype(jnp.float32)
    dinv = lax.rsqrt(deg)
    deg_max = jnp.max(deg)

    x = data
    p1 = dinv * x
    b1 = deg_max * jnp.max(jnp.abs(p1))
    z1 = dinv * (_fx_scatter(p1, b1, src3, dst3) + dinv * p1)
    h1a = jax.nn.relu(z1 @ a_cw1 + a_cb1)
    h1c = jax.nn.relu(z1 @ c_cw1 + c_cb1)

    q2 = jnp.concatenate([h1a @ a_cw2, h1c @ c_cw2], axis=1)
    p2 = dinv * q2
    b2 = deg_max * jnp.max(jnp.abs(p2))
    z2 = dinv * (_fx_scatter(p2, b2, src3, dst3) + dinv * p2)
    h2a = jax.nn.relu(z2[:, 0:2] + a_cb2)
    h2c = jax.nn.relu(z2[:, 2:4] + c_cb2)

    q3 = jnp.concatenate([h2a @ a_cw3, h2c @ c_cw3], axis=1)
    p3 = dinv * q3
    b3 = deg_max * jnp.max(jnp.abs(p3))
    z3 = dinv * (_fx_scatter(p3, b3, src3, dst3) + dinv * p3)
    ga = (z3[:, 0:1] + a_cb3).reshape(1, N)
    gc = (z3[:, 1:2] + c_cb3).reshape(1, N)

    t1a, t1c = _head_layer1(ga, gc, a_lw1, a_lb1, c_lw1, c_lb1)
    probs, value, act = _head_layer2(t1a, t1c, a_lw2, a_lb2, c_lw2, c_lb2)
    return (probs, value, act.reshape(()).astype(jnp.int32))


# speedup vs baseline: 1.2321x; 1.0852x over previous
import functools

import jax
import jax.numpy as jnp
from jax import lax
from jax.experimental import pallas as pl
from jax.experimental.pallas import tpu as pltpu
from jax.experimental.pallas import tpu_sc as plsc

N = 4096
E = 262144
NC, NS = 2, 16
NW = NC * NS
EPT = E // NW
CHUNK = 128
CH = EPT // CHUNK
GRP = 8


def _make_edge_pass(F):
    mesh = plsc.VectorSubcoreMesh(core_axis_name="c", subcore_axis_name="s")
    rpt = N // NS

    @functools.partial(
        pl.kernel,
        out_type=jax.ShapeDtypeStruct((NC, N, F), jnp.int32),
        mesh=mesh,
        compiler_params=pltpu.CompilerParams(use_tc_tiling_on_sc=False),
        scratch_types=[
            pltpu.VMEM((CH, CHUNK), jnp.int32),
            pltpu.VMEM((CH, CHUNK), jnp.int32),
            pltpu.VMEM((GRP, CHUNK, F), jnp.int32),
            pltpu.VMEM_SHARED((N, F), jnp.int32),
            pltpu.SemaphoreType.DMA,
            pltpu.SemaphoreType.DMA,
        ],
    )
    def edge_pass(table_hbm, src_hbm, dst_hbm, zeros_hbm, out_hbm,
                  src_v, dst_v, msg_v, acc_sh, gsem, ssem):
        cid = lax.axis_index("c")
        sid = lax.axis_index("s")
        wid = cid * NS + sid

        pltpu.sync_copy(src_hbm.at[wid], src_v)
        pltpu.sync_copy(dst_hbm.at[wid], dst_v)
        pltpu.sync_copy(zeros_hbm.at[pl.ds(sid * rpt, rpt)],
                        acc_sh.at[pl.ds(sid * rpt, rpt)])
        plsc.subcore_barrier()

        def group(g, carry):
            gds = []
            for i in range(GRP):
                j = g * GRP + i
                gds.append(pltpu.async_copy(
                    table_hbm.at[src_v.at[j]], msg_v.at[i], gsem))
            sds = []
            for i in range(GRP):
                j = g * GRP + i
                gds[i].wait()
                sds.append(pltpu.async_copy(
                    msg_v.at[i], acc_sh.at[dst_v.at[j]], ssem, add=True))
            for d in sds:
                d.wait()
            return carry

        lax.fori_loop(0, CH // GRP, group, 0, unroll=False)
        plsc.subcore_barrier()

        pltpu.sync_copy(acc_sh.at[pl.ds(sid * rpt, rpt)],
                        out_hbm.at[cid].at[pl.ds(sid * rpt, rpt)])

    return edge_pass


FP = 8
_edge_pass_8 = _make_edge_pass(FP)
_FXS = 2.0 ** 20


def _fx_scatter(p, bound, src3, dst3):
    F = p.shape[1]
    s1 = jnp.exp2(23.0 - jnp.ceil(jnp.log2(jnp.maximum(bound, 1e-30))))
    ps = p * s1
    hi = jnp.round(ps)
    lo = jnp.round((ps - hi) * _FXS)
    pad = jnp.zeros((N, 4 - F), jnp.float32)
    table = jnp.concatenate([hi, pad, lo, pad], axis=1).astype(jnp.int32)
    parts = _edge_pass_8(table, src3, dst3, jnp.zeros((N, FP), jnp.int32))
    hs = (parts[0, :, :F] + parts[1, :, :F]).astype(jnp.float32)
    ls = (parts[0, :, 4:4 + F] + parts[1, :, 4:4 + F]).astype(jnp.float32)
    return (hs + ls * (1.0 / _FXS)) * (1.0 / s1)


_BJ = 512
_BI = 512


def _lw1_body(ga_ref, gc_ref, aw_ref, ab_ref, cw_ref, cb_ref, ta_ref, tc_ref):
    ta_ref[...] = jnp.tanh(
        jnp.dot(ga_ref[...], aw_ref[...], preferred_element_type=jnp.float32)
        + ab_ref[...])
    tc_ref[...] = jnp.tanh(
        jnp.dot(gc_ref[...], cw_ref[...], preferred_element_type=jnp.float32)
        + cb_ref[...])


def _head_layer1(ga, gc, a_lw1, a_lb1, c_lw1, c_lb1):
    H = a_lw1.shape[1]
    grid = (H // _BJ,)
    return pl.pallas_call(
        _lw1_body,
        grid=grid,
        in_specs=[
            pl.BlockSpec((1, N), lambda j: (0, 0)),
            pl.BlockSpec((1, N), lambda j: (0, 0)),
            pl.BlockSpec((N, _BJ), lambda j: (0, j)),
            pl.BlockSpec((1, _BJ), lambda j: (0, j)),
            pl.BlockSpec((N, _BJ), lambda j: (0, j)),
            pl.BlockSpec((1, _BJ), lambda j: (0, j)),
        ],
        out_specs=[pl.BlockSpec((1, _BJ), lambda j: (0, j))] * 2,
        out_shape=[jax.ShapeDtypeStruct((1, H), jnp.float32)] * 2,
    )(ga, gc, a_lw1, a_lb1.reshape(1, -1), c_lw1, c_lb1.reshape(1, -1))


def _lw2_body(ta_ref, tc_ref, aw_ref, ab_ref, cw_ref, cb_ref,
              probs_ref, val_ref, act_ref, acc_ref):
    j = pl.program_id(0)
    nj = pl.num_programs(0)

    acc_ref[0, pl.ds(j * _BJ2, _BJ2)] = jnp.tanh(
        jnp.dot(ta_ref[...], aw_ref[...], preferred_element_type=jnp.float32)
        + ab_ref[...])[0]

    @pl.when(j == nj - 1)
    def _():
        outa = acc_ref[...]
        m = jnp.max(outa)
        e = jnp.exp(outa - m)
        probs_ref[...] = e / jnp.sum(e)
        val_ref[...] = jnp.tanh(
            jnp.dot(tc_ref[...], cw_ref[...],
                    preferred_element_type=jnp.float32) + cb_ref[...])
        idx = lax.broadcasted_iota(jnp.int32, outa.shape, 1)
        act_ref[0, 0] = jnp.min(jnp.where(outa == m, idx, jnp.int32(2**30)))


_BJ2 = 512


def _head_layer2(t1a, t1c, a_lw2, a_lb2, c_lw2, c_lb2):
    H = a_lw2.shape[0]
    grid = (N // _BJ2,)
    return pl.pallas_call(
        _lw2_body,
        grid=grid,
        in_specs=[
            pl.BlockSpec((1, H), lambda j: (0, 0)),
            pl.BlockSpec((1, H), lambda j: (0, 0)),
            pl.BlockSpec((H, _BJ2), lambda j: (0, j)),
            pl.BlockSpec((1, _BJ2), lambda j: (0, j)),
            pl.BlockSpec((H, 1), lambda j: (0, 0)),
            pl.BlockSpec((1, 1), lambda j: (0, 0)),
        ],
        out_specs=[
            pl.BlockSpec((1, N), lambda j: (0, 0)),
            pl.BlockSpec((1, 1), lambda j: (0, 0)),
            pl.BlockSpec(memory_space=pltpu.MemorySpace.SMEM),
        ],
        out_shape=[
            jax.ShapeDtypeStruct((1, N), jnp.float32),
            jax.ShapeDtypeStruct((1, 1), jnp.float32),
            jax.ShapeDtypeStruct((1, 1), jnp.int32),
        ],
        scratch_shapes=[
            pltpu.VMEM((1, N), jnp.float32),
        ],
    )(t1a, t1c, a_lw2, a_lb2.reshape(1, -1), c_lw2, c_lb2.reshape(1, -1))


def kernel(data, edge_index, a_cw1, a_cb1, a_cw2, a_cb2, a_cw3, a_cb3,
           a_lw1, a_lb1, a_lw2, a_lb2, c_cw1, c_cb1, c_cw2, c_cb2,
           c_cw3, c_cb3, c_lw1, c_lb1, c_lw2, c_lb2):
    ei = edge_index.astype(jnp.int32)
    src3 = ei[0].reshape(NW, CH, CHUNK)
    dst3 = ei[1].reshape(NW, CH, CHUNK)

    deg_tab = jnp.concatenate(
        [jnp.ones((N, 1), jnp.int32), jnp.zeros((N, FP - 1), jnp.int32)], axis=1)
    dparts = _edge_pass_8(deg_tab, dst3, dst3, jnp.zeros((N, FP), jnp.int32))
    deg = 1.0 + (dparts[0, :, :1] + dparts[1, :, :1]).astype(jnp.float32)
    dinv = lax.rsqrt(deg)
    deg_max = jnp.max(deg)

    x = data
    p1 = dinv * x
    b1 = deg_max * jnp.max(jnp.abs(p1))
    z1 = dinv * (_fx_scatter(p1, b1, src3, dst3) + dinv * p1)
    h1a = jax.nn.relu(z1 @ a_cw1 + a_cb1)
    h1c = jax.nn.relu(z1 @ c_cw1 + c_cb1)

    q2 = jnp.concatenate([h1a @ a_cw2, h1c @ c_cw2], axis=1)
    p2 = dinv * q2
    b2 = deg_max * jnp.max(jnp.abs(p2))
    z2 = dinv * (_fx_scatter(p2, b2, src3, dst3) + dinv * p2)
    h2a = jax.nn.relu(z2[:, 0:2] + a_cb2)
    h2c = jax.nn.relu(z2[:, 2:4] + c_cb2)

    q3 = jnp.concatenate([h2a @ a_cw3, h2c @ c_cw3], axis=1)
    p3 = dinv * q3
    b3 = deg_max * jnp.max(jnp.abs(p3))
    z3 = dinv * (_fx_scatter(p3, b3, src3, dst3) + dinv * p3)
    ga = (z3[:, 0:1] + a_cb3).reshape(1, N)
    gc = (z3[:, 1:2] + c_cb3).reshape(1, N)

    t1a, t1c = _head_layer1(ga, gc, a_lw1, a_lb1, c_lw1, c_lb1)
    probs, value, act = _head_layer2(t1a, t1c, a_lw2, a_lb2, c_lw2, c_lb2)
    return (probs, value, act.reshape(()).astype(jnp.int32))
